# Initial kernel scaffold; baseline (speedup 1.0000x reference)
import jax
import jax.numpy as jnp
from jax.experimental import pallas as pl


def kernel(positions, scales, rotations, colors, opacities, view_matrix):
    raise NotImplementedError("write your pallas kernel here")


# speedup vs baseline: 14.1570x; 14.1570x over previous
import jax
import jax.numpy as jnp
from jax.experimental import pallas as pl
from jax.experimental.pallas import tpu as pltpu

N_G = 1024
H_IMG = 128
W_IMG = 128
FX = 110.9
FY = 110.9
CX = 64.0
CY = 64.0
NEAR = 0.01
FAR = 100.0
MAX_RADIUS = 32.0


def _b16(x):
    return x.astype(jnp.bfloat16).astype(jnp.float32)


def _project_kernel(view_ref, px_ref, py_ref, pz_ref, sx_ref, sy_ref, sz_ref,
                    qw_ref, qx_ref, qy_ref, qz_ref, op_ref, out_ref):
    v = view_ref
    vb = [[_b16(v[i, j]) for j in range(4)] for i in range(4)]
    px = _b16(px_ref[...])
    py = _b16(py_ref[...])
    pz = _b16(pz_ref[...])
    pcx = vb[0][0] * px + vb[0][1] * py + vb[0][2] * pz + vb[0][3]
    pcy = vb[1][0] * px + vb[1][1] * py + vb[1][2] * pz + vb[1][3]
    pcz = vb[2][0] * px + vb[2][1] * py + vb[2][2] * pz + vb[2][3]
    depth = -pcz

    qw = qw_ref[...]
    qx = qx_ref[...]
    qy = qy_ref[...]
    qz = qz_ref[...]
    qn = jnp.sqrt(qw * qw + qx * qx + qy * qy + qz * qz) + 1e-12
    w = qw / qn
    x = qx / qn
    y = qy / qn
    z = qz / qn
    r = [[1 - 2 * y * y - 2 * z * z, 2 * x * y - 2 * w * z, 2 * x * z + 2 * w * y],
         [2 * x * y + 2 * w * z, 1 - 2 * x * x - 2 * z * z, 2 * y * z - 2 * w * x],
         [2 * x * z - 2 * w * y, 2 * y * z + 2 * w * x, 1 - 2 * x * x - 2 * y * y]]
    s = [_b16(sx_ref[...]), _b16(sy_ref[...]), _b16(sz_ref[...])]
    rc = [[vb[i][0] * _b16(r[0][j]) + vb[i][1] * _b16(r[1][j])
           + vb[i][2] * _b16(r[2][j]) for j in range(3)] for i in range(3)]
    rs = [[_b16(rc[i][j]) * s[j] for j in range(3)] for i in range(3)]
    rsb = [[_b16(rs[i][j]) for j in range(3)] for i in range(3)]
    cov = [[rsb[i][0] * rsb[j][0] + rsb[i][1] * rsb[j][1] + rsb[i][2] * rsb[j][2]
            for j in range(3)] for i in range(3)]

    zsafe = jnp.maximum(jnp.abs(pcz), 0.01) * jnp.sign(pcz + 1e-8)
    z2 = zsafe * zsafe
    j00 = FX / -zsafe
    j02 = FX * pcx / z2
    j11 = FY / zsafe
    j12 = FY * pcy / z2
    j00b = _b16(j00)
    j02b = _b16(j02)
    j11b = _b16(j11)
    j12b = _b16(j12)
    covb = [[_b16(cov[i][j]) for j in range(3)] for i in range(3)]
    t00 = j00b * covb[0][0] + j02b * covb[2][0]
    t01 = j00b * covb[0][1] + j02b * covb[2][1]
    t02 = j00b * covb[0][2] + j02b * covb[2][2]
    t10 = j11b * covb[1][0] + j12b * covb[2][0]
    t11 = j11b * covb[1][1] + j12b * covb[2][1]
    t12 = j11b * covb[1][2] + j12b * covb[2][2]
    a = _b16(t00) * j00b + _b16(t02) * j02b
    b = _b16(t01) * j11b + _b16(t02) * j12b
    c = _b16(t10) * j00b + _b16(t12) * j02b
    d = _b16(t11) * j11b + _b16(t12) * j12b

    u = FX * pcx / -zsafe + CX
    vv = FY * -pcy / -zsafe + CY
    trace = a + d
    det = jnp.maximum(a * d - b * c, 1e-6)
    disc = jnp.maximum(trace * trace - 4.0 * det, 0.0)
    max_eig = (trace + jnp.sqrt(disc)) / 2.0
    radii = jnp.minimum(3.0 * jnp.sqrt(jnp.maximum(max_eig, 1e-6)), MAX_RADIUS)

    vis = ((depth > NEAR) & (depth < FAR)
           & (u + radii > 0) & (u - radii < W_IMG)
           & (vv + radii > 0) & (vv - radii < H_IMG))

    ar = a + 0.3
    dr = d + 0.3
    br = b
    det_r = jnp.maximum(ar * dr - br * br, 1e-6)
    inv_a = dr / det_r
    inv_d = ar / det_r
    inv_b = -br / det_r
    oe = op_ref[...] * vis.astype(jnp.float32)

    out_ref[0] = depth
    out_ref[1] = u
    out_ref[2] = vv
    out_ref[3] = inv_a
    out_ref[4] = inv_b
    out_ref[5] = inv_d
    out_ref[6] = oe
    out_ref[7] = radii * radii


def _sort_kernel(dcol_ref, drow_ref, icol_ref, irow_ref, m_ref, out_ref):
    dcol = dcol_ref[...]
    drow = drow_ref[...]
    icol = icol_ref[...]
    irow = irow_ref[...]
    lt = jnp.where((dcol < drow) | ((dcol == drow) & (icol < irow)), 1.0, 0.0)
    rank = jnp.sum(lt, axis=0, keepdims=True)
    perm = jnp.where(icol == rank, 1.0, 0.0)
    out_ref[...] = jax.lax.dot_general(
        perm, m_ref[...], (((1,), (0,)), ((), ())),
        precision=jax.lax.Precision.HIGHEST,
        preferred_element_type=jnp.float32)


def _raster_kernel(ms_ref, out_ref, t_ref):
    ys = jax.lax.broadcasted_iota(
        jnp.int32, (H_IMG, W_IMG), 0).astype(jnp.float32) + 0.5
    xs = jax.lax.broadcasted_iota(
        jnp.int32, (H_IMG, W_IMG), 1).astype(jnp.float32) + 0.5
    t_ref[...] = jnp.ones((H_IMG, W_IMG), jnp.float32)
    out_ref[...] = jnp.zeros((3, H_IMG, W_IMG), jnp.float32)

    def body(g, carry):
        u = ms_ref[g, 0]
        v = ms_ref[g, 1]
        inv_a = ms_ref[g, 2]
        inv_b = ms_ref[g, 3]
        inv_d = ms_ref[g, 4]
        oe = ms_ref[g, 5]
        r2 = ms_ref[g, 6]
        c0 = ms_ref[g, 7]
        c1 = ms_ref[g, 8]
        c2 = ms_ref[g, 9]

        @pl.when(oe > 0.0)
        def _():
            dx = xs - u
            dy = ys - v
            dx2 = dx * dx
            dy2 = dy * dy
            power = -0.5 * (inv_a * dx2 + inv_d * dy2) - inv_b * (dx * dy)
            power = jnp.minimum(power, 0.0)
            gauss = jnp.exp(power)
            alpha = jnp.where(dx2 + dy2 <= r2, oe * gauss, 0.0)
            alpha = jnp.clip(alpha, 0.0, 0.99)
            tcur = t_ref[...]
            wgt = _b16(tcur * alpha)
            out_ref[0, :, :] = out_ref[0, :, :] + wgt * _b16(c0)
            out_ref[1, :, :] = out_ref[1, :, :] + wgt * _b16(c1)
            out_ref[2, :, :] = out_ref[2, :, :] + wgt * _b16(c2)
            t_ref[...] = tcur * (1.0 - alpha)

        return carry

    jax.lax.fori_loop(0, N_G, body, 0)


def kernel(positions, scales, rotations, colors, opacities, view_matrix):
    f32 = jnp.float32
    px = positions[:, 0].reshape(8, 128)
    py = positions[:, 1].reshape(8, 128)
    pz = positions[:, 2].reshape(8, 128)
    sx = scales[:, 0].reshape(8, 128)
    sy = scales[:, 1].reshape(8, 128)
    sz = scales[:, 2].reshape(8, 128)
    qw = rotations[:, 0].reshape(8, 128)
    qx = rotations[:, 1].reshape(8, 128)
    qy = rotations[:, 2].reshape(8, 128)
    qz = rotations[:, 3].reshape(8, 128)
    op = opacities.reshape(8, 128)

    proj = pl.pallas_call(
        _project_kernel,
        out_shape=jax.ShapeDtypeStruct((8, 8, 128), f32),
        in_specs=[pl.BlockSpec(memory_space=pltpu.SMEM)]
                 + [pl.BlockSpec(memory_space=pltpu.VMEM)] * 11,
    )(view_matrix, px, py, pz, sx, sy, sz, qw, qx, qy, qz, op)

    flat = proj.reshape(8, N_G)
    depth = flat[0]
    m = jnp.concatenate(
        [flat[1:8].T, colors, jnp.zeros((N_G, 6), f32)], axis=1)
    dcol = depth.reshape(N_G, 1)
    drow = depth.reshape(1, N_G)
    idx = jnp.arange(N_G, dtype=f32)
    icol = idx.reshape(N_G, 1)
    irow = idx.reshape(1, N_G)

    ms = pl.pallas_call(
        _sort_kernel,
        out_shape=jax.ShapeDtypeStruct((N_G, 16), f32),
    )(dcol, drow, icol, irow, m)

    img = pl.pallas_call(
        _raster_kernel,
        out_shape=jax.ShapeDtypeStruct((3, H_IMG, W_IMG), f32),
        in_specs=[pl.BlockSpec(memory_space=pltpu.SMEM)],
        scratch_shapes=[pltpu.VMEM((H_IMG, W_IMG), f32)],
    )(ms)
    return jnp.transpose(img, (1, 2, 0))
